# Initial kernel scaffold; baseline (speedup 1.0000x reference)
import jax
import jax.numpy as jnp
from jax.experimental import pallas as pl


def kernel(x, cji, valence_mask, cutoff_w, rb, shb, idx_i, idx_j, tri_idx_k, edge_idx_kj, edge_idx_ji, W_node, b_node, W_c1, W_c2, W_t1, b_t1, W_t2, b_t2, W_basis, W_n1, b_n1, W_n2, b_n2, W_out):
    raise NotImplementedError("write your pallas kernel here")


# speedup vs baseline: 4.8873x; 4.8873x over previous
import functools

import jax
import jax.numpy as jnp
from jax.experimental import pallas as pl
from jax.experimental.pallas import tpu as pltpu

N, E, T, NORB = 10000, 160000, 320000, 9
HD, CD, VD = 128, 16, 32

BN = 2000
BE_C = 1000
BE_P = 2000


def _silu(v):
    return v * jax.nn.sigmoid(v)


def _full(shape):
    return pl.BlockSpec(shape, lambda i: (0,) * len(shape))


def _prenode_body(x_ref, wn_ref, bn_ref, x1_ref, xks_ref):
    h = jnp.dot(x_ref[...], wn_ref[...], preferred_element_type=jnp.float32)
    h = h + bn_ref[...]
    x1_ref[...] = h[:, :VD]
    xks_ref[...] = jax.nn.sigmoid(h[:, VD:])


def _prenode(x, W_node, b_node):
    return pl.pallas_call(
        _prenode_body,
        grid=(N // BN,),
        in_specs=[
            pl.BlockSpec((BN, HD), lambda i: (i, 0)),
            _full((HD, 2 * VD)),
            _full((1, 2 * VD)),
        ],
        out_specs=[
            pl.BlockSpec((BN, VD), lambda i: (i, 0)),
            pl.BlockSpec((BN, VD), lambda i: (i, 0)),
        ],
        out_shape=[
            jax.ShapeDtypeStruct((N, VD), jnp.float32),
            jax.ShapeDtypeStruct((N, VD), jnp.float32),
        ],
    )(x, W_node, b_node.reshape(1, 2 * VD))


def _cmlp_body(cji_ref, rb_ref, cut_ref, wc1_ref, wc2_ref, cw_ref, s_ref):
    w1 = wc1_ref[...]
    w2 = wc2_ref[...]
    cut = cut_ref[...]
    s = jnp.zeros((BE_C, VD), jnp.float32)
    for d in range(NORB):
        cd = _silu(jnp.dot(_silu(jnp.dot(cji_ref[:, d, :], w1,
                                         preferred_element_type=jnp.float32)),
                           w2, preferred_element_type=jnp.float32))
        cwd = cd * (rb_ref[:, d:d + 1] * cut)
        cw_ref[:, d * VD:(d + 1) * VD] = cwd
        s = s + cwd
    s_ref[...] = s


def _cmlp(cji, rb, cutoff_w, W_c1, W_c2):
    return pl.pallas_call(
        _cmlp_body,
        grid=(E // BE_C,),
        in_specs=[
            pl.BlockSpec((BE_C, NORB, CD), lambda i: (i, 0, 0)),
            pl.BlockSpec((BE_C, NORB), lambda i: (i, 0)),
            pl.BlockSpec((BE_C, 1), lambda i: (i, 0)),
            _full((CD, VD)),
            _full((VD, VD)),
        ],
        out_specs=[
            pl.BlockSpec((BE_C, NORB * VD), lambda i: (i, 0)),
            pl.BlockSpec((BE_C, VD), lambda i: (i, 0)),
        ],
        out_shape=[
            jax.ShapeDtypeStruct((E, NORB * VD), jnp.float32),
            jax.ShapeDtypeStruct((E, VD), jnp.float32),
        ],
    )(cji, rb, cutoff_w.reshape(E, 1), W_c1, W_c2)


def _post_body(agg_ref, s_ref, nf_ref, wt1_ref, bt1_ref, wt2_ref, bt2_ref,
               wb_ref, wn1_ref, bn1_ref, wn2_ref, bn2_ref, msg_ref):
    tbw = _silu(jnp.dot(_silu(jnp.dot(agg_ref[...], wt1_ref[...],
                                      preferred_element_type=jnp.float32)
                              + bt1_ref[...]),
                        wt2_ref[...], preferred_element_type=jnp.float32)
                + bt2_ref[...])
    lcao = (1.0 + tbw) * s_ref[...]
    n2 = jnp.sum(lcao * lcao, axis=-1, keepdims=True)
    lcao = lcao * jax.lax.rsqrt(jnp.maximum(n2, 1e-24))
    lcao = jnp.dot(lcao, wb_ref[...], preferred_element_type=jnp.float32)
    nf = _silu(jnp.dot(_silu(jnp.dot(nf_ref[...], wn1_ref[...],
                                     preferred_element_type=jnp.float32)
                             + bn1_ref[...]),
                       wn2_ref[...], preferred_element_type=jnp.float32)
               + bn2_ref[...])
    msg_ref[...] = lcao * nf


def _post(agg_e, s, nf_in, W_t1, b_t1, W_t2, b_t2, W_basis,
          W_n1, b_n1, W_n2, b_n2):
    return pl.pallas_call(
        _post_body,
        grid=(E // BE_P,),
        in_specs=[
            pl.BlockSpec((BE_P, VD), lambda i: (i, 0)),
            pl.BlockSpec((BE_P, VD), lambda i: (i, 0)),
            pl.BlockSpec((BE_P, 2 * VD), lambda i: (i, 0)),
            _full((VD, VD)), _full((1, VD)),
            _full((VD, VD)), _full((1, VD)),
            _full((VD, VD)),
            _full((2 * VD, VD)), _full((1, VD)),
            _full((VD, VD)), _full((1, VD)),
        ],
        out_specs=pl.BlockSpec((BE_P, VD), lambda i: (i, 0)),
        out_shape=jax.ShapeDtypeStruct((E, VD), jnp.float32),
    )(agg_e, s, nf_in, W_t1, b_t1.reshape(1, VD), W_t2, b_t2.reshape(1, VD),
      W_basis, W_n1, b_n1.reshape(1, VD), W_n2, b_n2.reshape(1, VD))


def _final_body(x_ref, agg_ref, wo_ref, out_ref):
    out_ref[...] = x_ref[...] + jnp.dot(agg_ref[...], wo_ref[...],
                                        preferred_element_type=jnp.float32)


def _final(x, agg_n, W_out):
    return pl.pallas_call(
        _final_body,
        grid=(N // BN,),
        in_specs=[
            pl.BlockSpec((BN, HD), lambda i: (i, 0)),
            pl.BlockSpec((BN, VD), lambda i: (i, 0)),
            _full((VD, HD)),
        ],
        out_specs=pl.BlockSpec((BN, HD), lambda i: (i, 0)),
        out_shape=jax.ShapeDtypeStruct((N, HD), jnp.float32),
    )(x, agg_n, W_out)


def kernel(x, cji, valence_mask, cutoff_w, rb, shb, idx_i, idx_j, tri_idx_k,
           edge_idx_kj, edge_idx_ji, W_node, b_node, W_c1, W_c2, W_t1, b_t1,
           W_t2, b_t2, W_basis, W_n1, b_n1, W_n2, b_n2, W_out):
    x1, xks = _prenode(x, W_node, b_node)
    cw, s = _cmlp(cji, rb, cutoff_w, W_c1, W_c2)

    cw_g = cw[edge_idx_kj].reshape(T, NORB, VD)
    tbo = jnp.einsum('td,tdh->th', shb, cw_g)
    n2 = jnp.sum(tbo * tbo, axis=-1, keepdims=True)
    tbo = tbo * jax.lax.rsqrt(jnp.maximum(n2, 1e-24))
    tbw_t = tbo * xks[tri_idx_k]
    agg_e = jax.ops.segment_sum(tbw_t, edge_idx_ji, num_segments=E)

    nf_in = jnp.concatenate([x1[idx_i], x1[idx_j]], axis=-1)

    msg = _post(agg_e, s, nf_in, W_t1, b_t1, W_t2, b_t2, W_basis,
                W_n1, b_n1, W_n2, b_n2)

    agg_n = jax.ops.segment_sum(msg, idx_i, num_segments=N)
    return _final(x, agg_n, W_out)
